# Initial kernel scaffold; baseline (speedup 1.0000x reference)
import jax
import jax.numpy as jnp
from jax.experimental import pallas as pl


def kernel(categorical_features, continous_features, emb_table, W, b):
    raise NotImplementedError("write your pallas kernel here")


# speedup vs baseline: 9.3194x; 9.3194x over previous
import functools

import jax
import jax.numpy as jnp
from jax import lax
from jax.experimental import pallas as pl
from jax.experimental.pallas import tpu as pltpu
from jax.experimental.pallas import tpu_sc as plsc

B = 16384
CAT = 26
CONT = 13
D = 64
V = 100000

NC, NS = 2, 16
NW = NC * NS
RPW = B // NW
IPW = RPW * CAT
IDXW = 128
NROW = IPW // IDXW

ROWS_BLK = 2000


def _tc_body(emb_ref, wcat_ref, cont_ref, wcont_ref, b_ref,
             scores_ref, contpart_ref):
    scores_ref[...] = jnp.dot(emb_ref[...], wcat_ref[...],
                              preferred_element_type=jnp.float32)

    @pl.when(pl.program_id(0) == 0)
    def _():
        contpart_ref[...] = (
            jnp.dot(cont_ref[...], wcont_ref[...],
                    preferred_element_type=jnp.float32) + b_ref[0, 0])


_tc_call = pl.pallas_call(
    _tc_body,
    grid=(V // ROWS_BLK,),
    in_specs=[
        pl.BlockSpec((ROWS_BLK, D), lambda i: (i, 0)),
        pl.BlockSpec((D, CAT), lambda i: (0, 0)),
        pl.BlockSpec((B, CONT), lambda i: (0, 0)),
        pl.BlockSpec((CONT, 1), lambda i: (0, 0)),
        pl.BlockSpec((1, 1), lambda i: (0, 0)),
    ],
    out_specs=[
        pl.BlockSpec((ROWS_BLK, CAT), lambda i: (i, 0)),
        pl.BlockSpec((B, 1), lambda i: (0, 0)),
    ],
    out_shape=[
        jax.ShapeDtypeStruct((V, CAT), jnp.float32),
        jax.ShapeDtypeStruct((B, 1), jnp.float32),
    ],
)


def _sc_body(scores_hbm, fidx_hbm, contpart_hbm, out_hbm,
             idx_v, g_v, cont_v, out_v, sem):
    w = lax.axis_index("s") * NC + lax.axis_index("c")
    base = w * RPW

    pltpu.sync_copy(fidx_hbm.at[w], idx_v)
    pltpu.sync_copy(contpart_hbm.at[pl.ds(base, RPW)], cont_v)

    K = 8
    for j in range(K):
        pltpu.async_copy(scores_hbm.at[idx_v.at[j]], g_v.at[j], sem)

    def fire_drain(j, carry):
        pltpu.async_copy(scores_hbm.at[idx_v.at[j + K]], g_v.at[j + K], sem)
        pltpu.make_async_copy(scores_hbm.at[idx_v.at[j]],
                              g_v.at[j], sem).wait()
        return carry

    lax.fori_loop(0, NROW - K, fire_drain, 0)
    for j in range(NROW - K, NROW):
        pltpu.make_async_copy(scores_hbm.at[idx_v.at[j]],
                              g_v.at[j], sem).wait()

    for m in range(RPW // 16):
        r0 = m * 16
        q, p = divmod(r0, IDXW)
        acc = cont_v[pl.ds(r0, 16)]
        for c in range(CAT):
            acc = acc + g_v[(RPW // IDXW) * c + q, pl.ds(p, 16)]
        out_v[pl.ds(r0, 16)] = 1.0 / (1.0 + jnp.exp(-acc))

    pltpu.sync_copy(out_v, out_hbm.at[pl.ds(base, RPW)])


_sc_call = functools.partial(
    pl.kernel,
    out_type=jax.ShapeDtypeStruct((B,), jnp.float32),
    mesh=plsc.VectorSubcoreMesh(core_axis_name="c", subcore_axis_name="s"),
    scratch_types=[
        pltpu.VMEM((NROW, IDXW), jnp.int32),
        pltpu.VMEM((NROW, IDXW), jnp.float32),
        pltpu.VMEM((RPW,), jnp.float32),
        pltpu.VMEM((RPW,), jnp.float32),
        pltpu.SemaphoreType.DMA,
    ],
)(_sc_body)


def kernel(categorical_features, continous_features, emb_table, W, b):
    wcat = W[:CAT * D].reshape(CAT, D).T
    wcont = W[CAT * D:]
    scores, contpart = _tc_call(
        emb_table, wcat, continous_features, wcont,
        b.reshape(1, 1).astype(jnp.float32))

    fidx = categorical_features.astype(jnp.int32) * CAT + jnp.arange(
        CAT, dtype=jnp.int32)[None, :]
    fidx = fidx.reshape(NW, RPW, CAT).transpose(0, 2, 1).reshape(
        NW, NROW, IDXW)

    out = _sc_call(scores.reshape(-1), fidx, contpart.reshape(-1))
    return out.reshape(B, 1)


# speedup vs baseline: 13.7533x; 1.4758x over previous
import functools

import jax
import jax.numpy as jnp
from jax import lax
from jax.experimental import pallas as pl
from jax.experimental.pallas import tpu as pltpu
from jax.experimental.pallas import tpu_sc as plsc

B = 16384
CAT = 26
CONT = 13
D = 64
V = 100000
PITCH = 128

NC, NS = 2, 16
NW = NC * NS
RPW = B // NW
IPW = RPW * CAT
IDXW = 128
NROW = IPW // IDXW

ROWS_BLK = 10000


def _tc_body(emb_ref, wpad_ref, scores_ref):
    scores_ref[...] = jnp.dot(emb_ref[...], wpad_ref[...],
                              preferred_element_type=jnp.float32)


_tc_call = pl.pallas_call(
    _tc_body,
    grid=(V // ROWS_BLK,),
    in_specs=[
        pl.BlockSpec((ROWS_BLK, D), lambda i: (i, 0)),
        pl.BlockSpec((D, PITCH), lambda i: (0, 0)),
    ],
    out_specs=pl.BlockSpec((ROWS_BLK, PITCH), lambda i: (i, 0)),
    out_shape=jax.ShapeDtypeStruct((V, PITCH), jnp.float32),
)


def _sc_body(scores_hbm, fidx_hbm, cont_hbm, wb_hbm, out_hbm,
             idx_v, g_v, cont_v, wb_v, out_v, sem):
    w = lax.axis_index("s") * NC + lax.axis_index("c")
    base = w * RPW

    pltpu.sync_copy(fidx_hbm.at[w], idx_v)
    pltpu.sync_copy(cont_hbm.at[pl.ds(base, RPW)], cont_v)
    pltpu.sync_copy(wb_hbm, wb_v)

    for j in range(NROW):
        pltpu.async_copy(scores_hbm.at[idx_v.at[j]], g_v.at[j], sem)
    for j in range(NROW):
        pltpu.make_async_copy(scores_hbm.at[idx_v.at[j]],
                              g_v.at[j], sem).wait()

    iota = lax.iota(jnp.int32, 16)
    iota_cat = iota * CAT

    for m in range(RPW // 16):
        r0 = m * 16
        acc = wb_v[CONT, :]
        rows = iota + r0
        for j in range(CONT):
            cv = plsc.load_gather(cont_v, [rows, jnp.full((16,), j, jnp.int32)])
            acc = acc + cv * wb_v[j, :]
        for c in range(CAT):
            p = iota_cat + (r0 * CAT + c)
            sv = plsc.load_gather(g_v, [p >> 7, p & 127])
            acc = acc + sv
        out_v[pl.ds(r0, 16)] = 1.0 / (1.0 + jnp.exp(-acc))

    pltpu.sync_copy(out_v, out_hbm.at[pl.ds(base, RPW)])


_sc_call = functools.partial(
    pl.kernel,
    out_type=jax.ShapeDtypeStruct((B,), jnp.float32),
    mesh=plsc.VectorSubcoreMesh(core_axis_name="c", subcore_axis_name="s"),
    compiler_params=pltpu.CompilerParams(needs_layout_passes=False),
    scratch_types=[
        pltpu.VMEM((NROW, IDXW), jnp.int32),
        pltpu.VMEM((NROW, IDXW), jnp.float32),
        pltpu.VMEM((RPW, CONT), jnp.float32),
        pltpu.VMEM((CONT + 1, 16), jnp.float32),
        pltpu.VMEM((RPW,), jnp.float32),
        pltpu.SemaphoreType.DMA,
    ],
)(_sc_body)


def kernel(categorical_features, continous_features, emb_table, W, b):
    wcat = W[:CAT * D].reshape(CAT, D).T
    wpad = jnp.zeros((D, PITCH), jnp.float32).at[:, :CAT].set(wcat)
    scores = _tc_call(emb_table, wpad)

    fidx = categorical_features.astype(jnp.int32) * PITCH + jnp.arange(
        CAT, dtype=jnp.int32)[None, :]
    fidx = fidx.reshape(NW, NROW, IDXW)

    wb = jnp.concatenate([W[CAT * D:, 0], b]).astype(jnp.float32)
    wb = jnp.broadcast_to(wb[:, None], (CONT + 1, 16))

    out = _sc_call(scores.reshape(-1), fidx, continous_features, wb)
    return out.reshape(B, 1)


# speedup vs baseline: 18.5627x; 1.3497x over previous
import functools

import jax
import jax.numpy as jnp
from jax import lax
from jax.experimental import pallas as pl
from jax.experimental.pallas import tpu as pltpu
from jax.experimental.pallas import tpu_sc as plsc

B = 16384
CAT = 26
CONT = 13
D = 64
V = 100000
PITCH = 128

NC, NS = 2, 16
NW = NC * NS
RPW = B // NW
IPW = RPW * CAT
IDXW = 128
NROW = IPW // IDXW

CTAB = 32
VPAD = 100096
COLS_BLK = 4352


def _tc_body(wcat_ref, embt_ref, scores_ref):
    scores_ref[...] = jnp.dot(wcat_ref[...], embt_ref[...],
                              preferred_element_type=jnp.float32)


_tc_call = pl.pallas_call(
    _tc_body,
    grid=(VPAD // COLS_BLK,),
    in_specs=[
        pl.BlockSpec((CTAB, D), lambda i: (0, 0)),
        pl.BlockSpec((D, COLS_BLK), lambda i: (0, i)),
    ],
    out_specs=pl.BlockSpec((CTAB, COLS_BLK), lambda i: (0, i)),
    out_shape=jax.ShapeDtypeStruct((CTAB, VPAD), jnp.float32),
)


def _sc_body(scores_hbm, fidx_hbm, cont_hbm, wb_hbm, out_hbm,
             idx_v, g_v, cont_v, wb_v, out_v, sem):
    w = lax.axis_index("s") * NC + lax.axis_index("c")
    base = w * RPW

    pltpu.sync_copy(fidx_hbm.at[w], idx_v)
    pltpu.sync_copy(cont_hbm.at[pl.ds(base, RPW)], cont_v)
    pltpu.sync_copy(wb_hbm, wb_v)

    for j in range(NROW):
        pltpu.async_copy(scores_hbm.at[idx_v.at[j]], g_v.at[j], sem)
    for j in range(NROW):
        pltpu.make_async_copy(scores_hbm.at[idx_v.at[j]],
                              g_v.at[j], sem).wait()

    iota = lax.iota(jnp.int32, 16)
    iota_cat = iota * CAT

    for m in range(RPW // 16):
        r0 = m * 16
        acc = wb_v[CONT, :]
        rows = iota + r0
        for j in range(CONT):
            cv = plsc.load_gather(cont_v, [rows, jnp.full((16,), j, jnp.int32)])
            acc = acc + cv * wb_v[j, :]
        for c in range(CAT):
            p = iota_cat + (r0 * CAT + c)
            sv = plsc.load_gather(g_v, [p >> 7, p & 127])
            acc = acc + sv
        out_v[pl.ds(r0, 16)] = 1.0 / (1.0 + jnp.exp(-acc))

    pltpu.sync_copy(out_v, out_hbm.at[pl.ds(base, RPW)])


_sc_call = functools.partial(
    pl.kernel,
    out_type=jax.ShapeDtypeStruct((B,), jnp.float32),
    mesh=plsc.VectorSubcoreMesh(core_axis_name="c", subcore_axis_name="s"),
    compiler_params=pltpu.CompilerParams(needs_layout_passes=False),
    scratch_types=[
        pltpu.VMEM((NROW, IDXW), jnp.int32),
        pltpu.VMEM((NROW, IDXW), jnp.float32),
        pltpu.VMEM((RPW, CONT), jnp.float32),
        pltpu.VMEM((CONT + 1, 16), jnp.float32),
        pltpu.VMEM((RPW,), jnp.float32),
        pltpu.SemaphoreType.DMA,
    ],
)(_sc_body)


def kernel(categorical_features, continous_features, emb_table, W, b):
    wcat = W[:CAT * D].reshape(CAT, D)
    wcat32 = jnp.zeros((CTAB, D), jnp.float32).at[:CAT].set(wcat)
    scores = _tc_call(wcat32, emb_table.T)

    fidx = categorical_features.astype(jnp.int32) + jnp.arange(
        CAT, dtype=jnp.int32)[None, :] * VPAD
    fidx = fidx.reshape(NW, NROW, IDXW)

    wb = jnp.concatenate([W[CAT * D:, 0], b]).astype(jnp.float32)
    wb = jnp.broadcast_to(wb[:, None], (CONT + 1, 16))

    out = _sc_call(scores.reshape(-1), fidx, continous_features, wb)
    return out.reshape(B, 1)


# speedup vs baseline: 24.9369x; 1.3434x over previous
import functools

import jax
import jax.numpy as jnp
from jax import lax
from jax.experimental import pallas as pl
from jax.experimental.pallas import tpu as pltpu
from jax.experimental.pallas import tpu_sc as plsc

B = 16384
CAT = 26
CONT = 13
D = 64
V = 100000
PITCH = 128

NC, NS = 2, 16
NW = NC * NS
RPW = B // NW
IPW = RPW * CAT
IDXW = 128
NROW = IPW // IDXW

CTAB = 32
VPAD = 100096
COLS_BLK = 4352


def _tc_body(wcat_ref, embt_ref, scores_ref):
    scores_ref[...] = jnp.dot(wcat_ref[...], embt_ref[...],
                              preferred_element_type=jnp.float32)


_tc_call = pl.pallas_call(
    _tc_body,
    grid=(VPAD // COLS_BLK,),
    in_specs=[
        pl.BlockSpec((CTAB, D), lambda i: (0, 0)),
        pl.BlockSpec((D, COLS_BLK), lambda i: (0, i)),
    ],
    out_specs=pl.BlockSpec((CTAB, COLS_BLK), lambda i: (0, i)),
    out_shape=jax.ShapeDtypeStruct((CTAB, VPAD), jnp.float32),
)


def _sc_body(scores_hbm, fidx_hbm, cont_hbm, wb_hbm, out_hbm,
             idx_v, g_v, cont_v, wb_v, out_v, sem):
    w = lax.axis_index("s") * NC + lax.axis_index("c")
    base = w * RPW

    pltpu.sync_copy(fidx_hbm.at[:, pl.ds(base, RPW)], idx_v)
    for c in range(CAT):
        for q in range(RPW // IDXW):
            pltpu.async_copy(
                scores_hbm.at[idx_v.at[c, pl.ds(q * IDXW, IDXW)]],
                g_v.at[c, pl.ds(q * IDXW, IDXW)], sem)

    pltpu.sync_copy(cont_hbm.at[:, pl.ds(base, RPW)], cont_v)
    pltpu.sync_copy(wb_hbm, wb_v)
    wrow = [wb_v[j, :] for j in range(CONT + 1)]
    for m in range(RPW // 16):
        r0 = m * 16
        acc = wrow[CONT]
        for j in range(CONT):
            acc = acc + cont_v[j, pl.ds(r0, 16)] * wrow[j]
        out_v[pl.ds(r0, 16)] = acc

    for c in range(CAT):
        for q in range(RPW // IDXW):
            pltpu.make_async_copy(
                scores_hbm.at[idx_v.at[c, pl.ds(q * IDXW, IDXW)]],
                g_v.at[c, pl.ds(q * IDXW, IDXW)], sem).wait()
    for m in range(RPW // 16):
        r0 = m * 16
        acc = out_v[pl.ds(r0, 16)]
        for c in range(CAT):
            acc = acc + g_v[c, pl.ds(r0, 16)]
        out_v[pl.ds(r0, 16)] = 1.0 / (1.0 + jnp.exp(-acc))

    pltpu.sync_copy(out_v, out_hbm.at[pl.ds(base, RPW)])


_sc_call = functools.partial(
    pl.kernel,
    out_type=jax.ShapeDtypeStruct((B,), jnp.float32),
    mesh=plsc.VectorSubcoreMesh(core_axis_name="c", subcore_axis_name="s"),
    compiler_params=pltpu.CompilerParams(needs_layout_passes=False),
    scratch_types=[
        pltpu.VMEM((CAT, RPW), jnp.int32),
        pltpu.VMEM((CAT, RPW), jnp.float32),
        pltpu.VMEM((CONT, RPW), jnp.float32),
        pltpu.VMEM((CONT + 1, 16), jnp.float32),
        pltpu.VMEM((RPW,), jnp.float32),
        pltpu.SemaphoreType.DMA,
    ],
)(_sc_body)


def kernel(categorical_features, continous_features, emb_table, W, b):
    wcat = W[:CAT * D].reshape(CAT, D)
    wcat32 = jnp.zeros((CTAB, D), jnp.float32).at[:CAT].set(wcat)
    scores = _tc_call(wcat32, emb_table.T)

    fidx = categorical_features.T.astype(jnp.int32) + jnp.arange(
        CAT, dtype=jnp.int32)[:, None] * VPAD

    wb = jnp.concatenate([W[CAT * D:, 0], b]).astype(jnp.float32)
    wb = jnp.broadcast_to(wb[:, None], (CONT + 1, 16))

    out = _sc_call(scores.reshape(-1), fidx, continous_features.T, wb)
    return out.reshape(B, 1)
